# baseline (speedup 1.0000x reference)
import jax
import jax.numpy as jnp
from jax import lax
from jax.experimental import pallas as pl
from jax.experimental.pallas import tpu as pltpu

N1 = 4096
N2 = 4096
D = 256
BM = 2048
NSTEP = N1 // BM


def _body(d0_ref, d1_ref, oi_ref, os_ref, nn12_s, sc_s, cmax_s, carg_s):
    i = pl.program_id(0)
    sim = jax.lax.dot_general(
        d0_ref[0], d1_ref[0],
        dimension_numbers=(((0,), (0,)), ((), ())),
        preferred_element_type=jnp.float32,
    )

    LC = 128
    RS = 64
    for rs in range(BM // RS):
        r0 = rs * RS
        rv = sim[r0:r0 + RS, 0:LC]
        rc = jnp.zeros((RS, LC), jnp.int32)
        for c in range(1, N2 // LC):
            v = sim[r0:r0 + RS, c * LC:(c + 1) * LC]
            m = v > rv
            rv = jnp.where(m, v, rv)
            rc = jnp.where(m, c, rc)
        rmax = jnp.max(rv, axis=1, keepdims=True)
        cand_j = rc * LC + lax.broadcasted_iota(jnp.int32, (RS, LC), 1)
        rarg = jnp.min(jnp.where(rv == rmax, cand_j, 2**30),
                       axis=1, keepdims=True)
        nn12_s[pl.ds(i * BM + r0, RS), :] = rarg
        sc_s[pl.ds(i * BM + r0, RS), :] = rmax

    SC_ = 8
    LS = 1024
    bmax_parts = []
    barg_parts = []
    for ls in range(N2 // LS):
        l0 = ls * LS
        cv = sim[0:SC_, l0:l0 + LS]
        cc = jnp.zeros((SC_, LS), jnp.int32)
        for c in range(1, BM // SC_):
            v = sim[c * SC_:(c + 1) * SC_, l0:l0 + LS]
            m = v > cv
            cv = jnp.where(m, v, cv)
            cc = jnp.where(m, c, cc)
        bm_ = jnp.max(cv, axis=0, keepdims=True)
        cand_i = (cc * SC_ + lax.broadcasted_iota(jnp.int32, (SC_, LS), 0)
                  + i * BM)
        ba_ = jnp.min(jnp.where(cv == bm_, cand_i, 2**30),
                      axis=0, keepdims=True)
        bmax_parts.append(bm_)
        barg_parts.append(ba_)
    bmax = jnp.concatenate(bmax_parts, axis=1)
    barg = jnp.concatenate(barg_parts, axis=1)

    @pl.when(i == 0)
    def _init():
        cmax_s[...] = bmax
        carg_s[...] = barg

    @pl.when(i > 0)
    def _update():
        prev_max = cmax_s[...]
        prev_arg = carg_s[...]
        better = bmax > prev_max
        cmax_s[...] = jnp.where(better, bmax, prev_max)
        carg_s[...] = jnp.where(better, barg, prev_arg)

    @pl.when(i == NSTEP - 1)
    def _final():
        nn12 = nn12_s[...]
        carg = carg_s[...]
        tbl = jnp.reshape(carg.astype(jnp.float32), (32, 128))
        hi = nn12 // 128
        lo = nn12 - hi * 128
        oh = (lax.broadcasted_iota(jnp.int32, (N1, 32), 1) == hi
              ).astype(jnp.float32)
        s = jax.lax.dot_general(
            oh, tbl, dimension_numbers=(((1,), (0,)), ((), ())),
            precision=jax.lax.Precision.HIGHEST,
            preferred_element_type=jnp.float32,
        )
        lane = lax.broadcasted_iota(jnp.int32, (N1, 128), 1)
        g = jnp.sum(jnp.where(lane == lo, s, 0.0), axis=1, keepdims=True)
        ids = lax.broadcasted_iota(jnp.int32, (N1, 1), 0)
        mut = g == ids.astype(jnp.float32)
        oi = jnp.where(mut, nn12, jnp.int32(-1))
        os = jnp.where(mut, sc_s[...], jnp.float32(-1.0))
        oi_ref[...] = jnp.reshape(oi, (1, N1))
        os_ref[...] = jnp.reshape(os, (1, N1))


def _run(d0, d1):
    return pl.pallas_call(
        _body,
        grid=(NSTEP,),
        in_specs=[
            pl.BlockSpec((1, D, BM), lambda i: (0, 0, i)),
            pl.BlockSpec((1, D, N2), lambda i: (0, 0, 0)),
        ],
        out_specs=[
            pl.BlockSpec((1, N1), lambda i: (0, 0)),
            pl.BlockSpec((1, N1), lambda i: (0, 0)),
        ],
        out_shape=[
            jax.ShapeDtypeStruct((1, N1), jnp.int32),
            jax.ShapeDtypeStruct((1, N1), jnp.float32),
        ],
        scratch_shapes=[
            pltpu.VMEM((N1, 1), jnp.int32),
            pltpu.VMEM((N1, 1), jnp.float32),
            pltpu.VMEM((1, N2), jnp.float32),
            pltpu.VMEM((1, N2), jnp.int32),
        ],
    )(d0, d1)


def kernel(descriptors0, descriptors1, keypoints0, keypoints1):
    indices0, mscores0 = _run(descriptors0, descriptors1)
    return indices0, indices0, mscores0, mscores0


# speedup vs baseline: 1.0500x; 1.0500x over previous
import jax
import jax.numpy as jnp
from jax import lax
from jax.experimental import pallas as pl
from jax.experimental.pallas import tpu as pltpu

N1 = 4096
N2 = 4096
D = 256
BM = 2048
NSTEP = N1 // BM


def _body(d0_ref, d1_ref, oi_ref, os_ref, nn12_s, sc_s, cmax_s, carg_s):
    i = pl.program_id(0)
    sim = jax.lax.dot_general(
        d0_ref[0], d1_ref[0],
        dimension_numbers=(((0,), (0,)), ((), ())),
        preferred_element_type=jnp.float32,
    )

    LC = 128
    RS = 64
    for rs in range(BM // RS):
        r0 = rs * RS
        rv = sim[r0:r0 + RS, 0:LC]
        rc = jnp.zeros((RS, LC), jnp.int32)
        for c in range(1, N2 // LC):
            v = sim[r0:r0 + RS, c * LC:(c + 1) * LC]
            m = v > rv
            rv = jnp.where(m, v, rv)
            rc = jnp.where(m, c, rc)
        rmax = jnp.max(rv, axis=1, keepdims=True)
        cand_j = rc * LC + lax.broadcasted_iota(jnp.int32, (RS, LC), 1)
        rarg = jnp.min(jnp.where(rv == rmax, cand_j, 2**30),
                       axis=1, keepdims=True)
        nn12_s[pl.ds(i * BM + r0, RS), :] = rarg
        sc_s[pl.ds(i * BM + r0, RS), :] = rmax

    SC_ = 8
    LS = 1024
    bmax_parts = []
    barg_parts = []
    for ls in range(N2 // LS):
        l0 = ls * LS
        cv = sim[0:SC_, l0:l0 + LS]
        cc = jnp.zeros((SC_, LS), jnp.int32)
        for c in range(1, BM // SC_):
            v = sim[c * SC_:(c + 1) * SC_, l0:l0 + LS]
            m = v > cv
            cv = jnp.where(m, v, cv)
            cc = jnp.where(m, c, cc)
        bm_ = jnp.max(cv, axis=0, keepdims=True)
        cand_i = (cc * SC_ + lax.broadcasted_iota(jnp.int32, (SC_, LS), 0)
                  + i * BM)
        ba_ = jnp.min(jnp.where(cv == bm_, cand_i, 2**30),
                      axis=0, keepdims=True)
        bmax_parts.append(bm_)
        barg_parts.append(ba_)
    bmax = jnp.concatenate(bmax_parts, axis=1)
    barg = jnp.concatenate(barg_parts, axis=1)

    @pl.when(i == 0)
    def _init():
        cmax_s[...] = bmax
        carg_s[...] = barg

    @pl.when(i > 0)
    def _update():
        prev_max = cmax_s[...]
        prev_arg = carg_s[...]
        better = bmax > prev_max
        cmax_s[...] = jnp.where(better, bmax, prev_max)
        carg_s[...] = jnp.where(better, barg, prev_arg)

    @pl.when(i == NSTEP - 1)
    def _final():
        nn12 = nn12_s[...]
        carg = carg_s[...]
        tbl_i = jnp.reshape(carg, (32, 128))
        tbl_h = (tbl_i >> 6).astype(jnp.float32)
        tbl_l = (tbl_i & 63).astype(jnp.float32)
        hi = nn12 // 128
        lo = nn12 - hi * 128
        oh = (lax.broadcasted_iota(jnp.int32, (N1, 32), 1) == hi
              ).astype(jnp.float32)
        dn = (((1,), (0,)), ((), ()))
        s_h = jax.lax.dot_general(oh, tbl_h, dimension_numbers=dn,
                                  preferred_element_type=jnp.float32)
        s_l = jax.lax.dot_general(oh, tbl_l, dimension_numbers=dn,
                                  preferred_element_type=jnp.float32)
        s = s_h * 64.0 + s_l
        lane = lax.broadcasted_iota(jnp.int32, (N1, 128), 1)
        g = jnp.sum(jnp.where(lane == lo, s, 0.0), axis=1, keepdims=True)
        ids = lax.broadcasted_iota(jnp.int32, (N1, 1), 0)
        mut = g == ids.astype(jnp.float32)
        oi = jnp.where(mut, nn12, jnp.int32(-1))
        os = jnp.where(mut, sc_s[...], jnp.float32(-1.0))
        oi_ref[...] = jnp.reshape(oi, (1, N1))
        os_ref[...] = jnp.reshape(os, (1, N1))


def _run(d0, d1):
    return pl.pallas_call(
        _body,
        grid=(NSTEP,),
        in_specs=[
            pl.BlockSpec((1, D, BM), lambda i: (0, 0, i)),
            pl.BlockSpec((1, D, N2), lambda i: (0, 0, 0)),
        ],
        out_specs=[
            pl.BlockSpec((1, N1), lambda i: (0, 0)),
            pl.BlockSpec((1, N1), lambda i: (0, 0)),
        ],
        out_shape=[
            jax.ShapeDtypeStruct((1, N1), jnp.int32),
            jax.ShapeDtypeStruct((1, N1), jnp.float32),
        ],
        scratch_shapes=[
            pltpu.VMEM((N1, 1), jnp.int32),
            pltpu.VMEM((N1, 1), jnp.float32),
            pltpu.VMEM((1, N2), jnp.float32),
            pltpu.VMEM((1, N2), jnp.int32),
        ],
    )(d0, d1)


def kernel(descriptors0, descriptors1, keypoints0, keypoints1):
    indices0, mscores0 = _run(descriptors0, descriptors1)
    return indices0, indices0, mscores0, mscores0
